# Initial kernel scaffold; baseline (speedup 1.0000x reference)
import jax
import jax.numpy as jnp
from jax.experimental import pallas as pl


def kernel(data, edge_index, W1, b1, g1, be1, W2, b2, g2, be2, W3, b3):
    raise NotImplementedError("write your pallas kernel here")


# speedup vs baseline: 16.8129x; 16.8129x over previous
import functools

import jax
import jax.numpy as jnp
from jax import lax
from jax.experimental import pallas as pl
from jax.experimental.pallas import tpu as pltpu
from jax.experimental.pallas import tpu_sc as plsc

N = 10000
D_IN = 128
H = 64
C = 40

NC = 2
NS = 16
NW = NC * NS
CHUNK = 128
N_PAD = 10240
RPT = N_PAD // NS

_mesh = functools.partial(
    plsc.VectorSubcoreMesh, core_axis_name="c", subcore_axis_name="s"
)


def _zero_buf(buf, nrow, width):
  z = jnp.zeros((16,), jnp.float32)

  def body(i, _):
    for k in range(width // 16):
      buf[i, pl.ds(16 * k, 16)] = z
    return 0

  lax.fori_loop(0, nrow, body, 0)


def _make_deg_kernel(chunks):
  @functools.partial(
      pl.kernel,
      out_type=jax.ShapeDtypeStruct((NC, N_PAD, 16), jnp.float32),
      mesh=_mesh(),
      scratch_types=[
          pltpu.VMEM((chunks, CHUNK), jnp.int32),
          pltpu.VMEM((CHUNK, 16), jnp.float32),
          pltpu.VMEM((CHUNK, 16), jnp.float32),
          pltpu.VMEM_SHARED((N_PAD, 16), jnp.float32),
      ],
  )
  def deg_kernel(cols_hbm, out_hbm, cols_v, ones_v, stage_v, deg_sh):
    cid = lax.axis_index("c")
    sid = lax.axis_index("s")
    wid = sid * NC + cid

    one = jnp.ones((16,), jnp.float32)

    def fill(i, _):
      ones_v[i, pl.ds(0, 16)] = one
      return 0

    lax.fori_loop(0, CHUNK, fill, 0)
    _zero_buf(stage_v, CHUNK, 16)

    for j in range(RPT // CHUNK):
      pltpu.sync_copy(stage_v, deg_sh.at[pl.ds(sid * RPT + j * CHUNK, CHUNK)])
    plsc.subcore_barrier()

    pltpu.sync_copy(cols_hbm.at[wid], cols_v)

    def body(j, _):
      pltpu.sync_copy(ones_v, deg_sh.at[cols_v.at[j]], add=True)
      return 0

    lax.fori_loop(0, chunks, body, 0)
    plsc.subcore_barrier()

    for j in range(RPT // CHUNK):
      r0 = sid * RPT + j * CHUNK
      pltpu.sync_copy(deg_sh.at[pl.ds(r0, CHUNK)], stage_v)
      pltpu.sync_copy(stage_v, out_hbm.at[cid, pl.ds(r0, CHUNK)])

  return deg_kernel


def _make_prop_kernel(chunks):
  @functools.partial(
      pl.kernel,
      out_type=jax.ShapeDtypeStruct((NC, N_PAD, H), jnp.float32),
      mesh=_mesh(),
      scratch_types=[
          pltpu.VMEM((chunks, CHUNK), jnp.int32),
          pltpu.VMEM((chunks, CHUNK), jnp.int32),
          pltpu.VMEM((CHUNK, H), jnp.float32),
          pltpu.VMEM((CHUNK, H), jnp.float32),
          pltpu.VMEM_SHARED((N_PAD, H), jnp.float32),
          pltpu.SemaphoreType.DMA,
          pltpu.SemaphoreType.DMA,
      ],
      compiler_params=pltpu.CompilerParams(use_tc_tiling_on_sc=False),
  )
  def prop_kernel(hp_hbm, rows_hbm, cols_hbm, out_hbm, rows_v, cols_v,
                  buf_a, buf_b, acc_sh, sem_a, sem_b):
    cid = lax.axis_index("c")
    sid = lax.axis_index("s")
    wid = sid * NC + cid

    _zero_buf(buf_a, CHUNK, H)
    for j in range(RPT // CHUNK):
      pltpu.sync_copy(buf_a, acc_sh.at[pl.ds(sid * RPT + j * CHUNK, CHUNK)])
    plsc.subcore_barrier()

    pltpu.sync_copy(rows_hbm.at[wid], rows_v)
    pltpu.sync_copy(cols_hbm.at[wid], cols_v)

    cp_a = pltpu.async_copy(hp_hbm.at[rows_v.at[0]], buf_a, sem_a)

    def body(j, _):
      cp_b = pltpu.async_copy(hp_hbm.at[rows_v.at[2 * j + 1]], buf_b, sem_b)
      cp_a.wait()
      pltpu.sync_copy(buf_a, acc_sh.at[cols_v.at[2 * j]], add=True)
      pltpu.async_copy(hp_hbm.at[rows_v.at[2 * j + 2]], buf_a, sem_a)
      cp_b.wait()
      pltpu.sync_copy(buf_b, acc_sh.at[cols_v.at[2 * j + 1]], add=True)
      return 0

    lax.fori_loop(0, chunks // 2 - 1, body, 0)
    j = chunks - 2
    cp_b = pltpu.async_copy(hp_hbm.at[rows_v.at[j + 1]], buf_b, sem_b)
    cp_a.wait()
    pltpu.sync_copy(buf_a, acc_sh.at[cols_v.at[j]], add=True)
    cp_b.wait()
    pltpu.sync_copy(buf_b, acc_sh.at[cols_v.at[j + 1]], add=True)
    plsc.subcore_barrier()

    for j in range(RPT // CHUNK):
      r0 = sid * RPT + j * CHUNK
      pltpu.sync_copy(acc_sh.at[pl.ds(r0, CHUNK)], buf_a)
      pltpu.sync_copy(buf_a, out_hbm.at[cid, pl.ds(r0, CHUNK)])

  return prop_kernel


def _tc1(data_ref, w1_ref, degp_ref, h1p_ref, dinv_ref):
  deg = degp_ref[0, :N, 0:1] + degp_ref[1, :N, 0:1] + 1.0
  dinv = lax.rsqrt(deg)
  h1 = jnp.dot(data_ref[...], w1_ref[...],
               preferred_element_type=jnp.float32,
               precision=lax.Precision.HIGHEST)
  h1p_ref[...] = h1 * dinv
  dinv_ref[...] = dinv


def _tc2(sp_ref, h1p_ref, dinv_ref, b1_ref, g1_ref, be1_ref, w2_ref,
         h2p_ref):
  dinv = dinv_ref[...]
  s = sp_ref[0, :N, :] + sp_ref[1, :N, :] + h1p_ref[...]
  x = jax.nn.relu(dinv * s + b1_ref[...])
  m = jnp.mean(x, axis=0, keepdims=True)
  v = jnp.mean((x - m) ** 2, axis=0, keepdims=True)
  y = (x - m) * lax.rsqrt(v + 1e-5) * g1_ref[...] + be1_ref[...]
  y = jax.nn.relu(y)
  h2 = jnp.dot(y, w2_ref[...], preferred_element_type=jnp.float32,
               precision=lax.Precision.HIGHEST)
  h2p_ref[...] = h2 * dinv


def _tc3(sp_ref, h2p_ref, dinv_ref, b2_ref, g2_ref, be2_ref, w3_ref, b3_ref,
         out_ref):
  dinv = dinv_ref[...]
  s = sp_ref[0, :N, :] + sp_ref[1, :N, :] + h2p_ref[...]
  x = jax.nn.relu(dinv * s + b2_ref[...])
  m = jnp.mean(x, axis=0, keepdims=True)
  v = jnp.mean((x - m) ** 2, axis=0, keepdims=True)
  y = (x - m) * lax.rsqrt(v + 1e-5) * g2_ref[...] + be2_ref[...]
  z = jnp.dot(y, w3_ref[...], preferred_element_type=jnp.float32,
              precision=lax.Precision.HIGHEST)
  out_ref[...] = jax.nn.relu(z + b3_ref[...])


@jax.jit
def kernel(data, edge_index, W1, b1, g1, be1, W2, b2, g2, be2, W3, b3):
  E = edge_index.shape[1]
  epw = -(-E // (NW * CHUNK)) * CHUNK
  if (epw // CHUNK) % 2:
    epw += CHUNK
  chunks = epw // CHUNK
  e_pad = NW * epw

  row = edge_index[0]
  col = edge_index[1]
  rows3 = jnp.concatenate(
      [row, jnp.zeros((e_pad - E,), row.dtype)]).reshape(NW, chunks, CHUNK)
  cols3 = jnp.concatenate(
      [col, jnp.full((e_pad - E,), N, col.dtype)]).reshape(NW, chunks, CHUNK)

  degp = _make_deg_kernel(chunks)(cols3)

  h1p, dinv = pl.pallas_call(
      _tc1,
      out_shape=[
          jax.ShapeDtypeStruct((N, H), jnp.float32),
          jax.ShapeDtypeStruct((N, 1), jnp.float32),
      ],
  )(data, W1, degp)

  prop = _make_prop_kernel(chunks)
  s1p = prop(h1p, rows3, cols3)

  h2p = pl.pallas_call(
      _tc2,
      out_shape=jax.ShapeDtypeStruct((N, H), jnp.float32),
  )(s1p, h1p, dinv, b1.reshape(1, H), g1.reshape(1, H), be1.reshape(1, H), W2)

  s2p = prop(h2p, rows3, cols3)

  out = pl.pallas_call(
      _tc3,
      out_shape=jax.ShapeDtypeStruct((N, C), jnp.float32),
  )(s2p, h2p, dinv, b2.reshape(1, H), g2.reshape(1, H), be2.reshape(1, H),
    W3, b3.reshape(1, C))

  return out


# speedup vs baseline: 32.6322x; 1.9409x over previous
import functools

import jax
import jax.numpy as jnp
from jax import lax
from jax.experimental import pallas as pl
from jax.experimental.pallas import tpu as pltpu
from jax.experimental.pallas import tpu_sc as plsc

N = 10000
D_IN = 128
H = 64
C = 40

NC = 2
NS = 16
NW = NC * NS
CHUNK = 128
N_PAD = 10240
RPT = N_PAD // NS

_mesh = functools.partial(
    plsc.VectorSubcoreMesh, core_axis_name="c", subcore_axis_name="s"
)


def _zero_buf(buf, nrow, width):
  z = jnp.zeros((16,), jnp.float32)

  def body(i, _):
    for k in range(width // 16):
      buf[i, pl.ds(16 * k, 16)] = z
    return 0

  lax.fori_loop(0, nrow, body, 0)


def _make_deg_kernel(chunks):
  @functools.partial(
      pl.kernel,
      out_type=jax.ShapeDtypeStruct((NC, N_PAD, 16), jnp.float32),
      mesh=_mesh(),
      scratch_types=[
          pltpu.VMEM((chunks, CHUNK), jnp.int32),
          pltpu.VMEM((CHUNK, 16), jnp.float32),
          pltpu.VMEM((CHUNK, 16), jnp.float32),
          pltpu.VMEM_SHARED((N_PAD, 16), jnp.float32),
      ],
  )
  def deg_kernel(cols_hbm, out_hbm, cols_v, ones_v, stage_v, deg_sh):
    cid = lax.axis_index("c")
    sid = lax.axis_index("s")
    wid = sid * NC + cid

    one = jnp.ones((16,), jnp.float32)

    def fill(i, _):
      ones_v[i, pl.ds(0, 16)] = one
      return 0

    lax.fori_loop(0, CHUNK, fill, 0)
    _zero_buf(stage_v, CHUNK, 16)

    for j in range(RPT // CHUNK):
      pltpu.sync_copy(stage_v, deg_sh.at[pl.ds(sid * RPT + j * CHUNK, CHUNK)])
    plsc.subcore_barrier()

    pltpu.sync_copy(cols_hbm.at[wid], cols_v)

    def body(j, _):
      pltpu.sync_copy(ones_v, deg_sh.at[cols_v.at[j]], add=True)
      return 0

    lax.fori_loop(0, chunks, body, 0)
    plsc.subcore_barrier()

    for j in range(RPT // CHUNK):
      r0 = sid * RPT + j * CHUNK
      pltpu.sync_copy(deg_sh.at[pl.ds(r0, CHUNK)], stage_v)
      pltpu.sync_copy(stage_v, out_hbm.at[cid, pl.ds(r0, CHUNK)])

  return deg_kernel


def _make_prop_kernel(chunks):
  @functools.partial(
      pl.kernel,
      out_type=jax.ShapeDtypeStruct((NC, N_PAD, H), jnp.float32),
      mesh=_mesh(),
      scratch_types=[
          pltpu.VMEM((chunks, CHUNK), jnp.int32),
          pltpu.VMEM((chunks, CHUNK), jnp.int32),
          pltpu.VMEM((CHUNK, H), jnp.float32),
          pltpu.VMEM((CHUNK, H), jnp.float32),
          pltpu.VMEM_SHARED((N_PAD, H), jnp.float32),
          pltpu.VMEM_SHARED((N_PAD, H), jnp.float32),
          pltpu.SemaphoreType.DMA,
          pltpu.SemaphoreType.DMA,
      ],
      compiler_params=pltpu.CompilerParams(use_tc_tiling_on_sc=False),
  )
  def prop_kernel(hp_hbm, rows_hbm, cols_hbm, out_hbm, rows_v, cols_v,
                  buf_a, buf_b, acc_sh, tab_sh, sem_a, sem_b):
    cid = lax.axis_index("c")
    sid = lax.axis_index("s")
    wid = sid * NC + cid

    for j in range(RPT // CHUNK):
      r0 = sid * RPT + j * CHUNK
      pltpu.sync_copy(hp_hbm.at[pl.ds(r0, CHUNK)], buf_b)
      pltpu.sync_copy(buf_b, tab_sh.at[pl.ds(r0, CHUNK)])
    _zero_buf(buf_a, CHUNK, H)
    for j in range(RPT // CHUNK):
      pltpu.sync_copy(buf_a, acc_sh.at[pl.ds(sid * RPT + j * CHUNK, CHUNK)])
    plsc.subcore_barrier()

    pltpu.sync_copy(rows_hbm.at[wid], rows_v)
    pltpu.sync_copy(cols_hbm.at[wid], cols_v)

    cp_a = pltpu.async_copy(tab_sh.at[rows_v.at[0]], buf_a, sem_a)

    def body(j, _):
      cp_b = pltpu.async_copy(tab_sh.at[rows_v.at[2 * j + 1]], buf_b, sem_b)
      cp_a.wait()
      pltpu.sync_copy(buf_a, acc_sh.at[cols_v.at[2 * j]], add=True)
      pltpu.async_copy(tab_sh.at[rows_v.at[2 * j + 2]], buf_a, sem_a)
      cp_b.wait()
      pltpu.sync_copy(buf_b, acc_sh.at[cols_v.at[2 * j + 1]], add=True)
      return 0

    lax.fori_loop(0, chunks // 2 - 1, body, 0)
    j = chunks - 2
    cp_b = pltpu.async_copy(tab_sh.at[rows_v.at[j + 1]], buf_b, sem_b)
    cp_a.wait()
    pltpu.sync_copy(buf_a, acc_sh.at[cols_v.at[j]], add=True)
    cp_b.wait()
    pltpu.sync_copy(buf_b, acc_sh.at[cols_v.at[j + 1]], add=True)
    plsc.subcore_barrier()

    for j in range(RPT // CHUNK):
      r0 = sid * RPT + j * CHUNK
      pltpu.sync_copy(acc_sh.at[pl.ds(r0, CHUNK)], buf_a)
      pltpu.sync_copy(buf_a, out_hbm.at[cid, pl.ds(r0, CHUNK)])

  return prop_kernel


def _tc1(data_ref, w1_ref, degp_ref, h1p_ref, dinv_ref):
  deg = degp_ref[0, :N, 0:1] + degp_ref[1, :N, 0:1] + 1.0
  dinv = lax.rsqrt(deg)
  h1 = jnp.dot(data_ref[...], w1_ref[...],
               preferred_element_type=jnp.float32,
               precision=lax.Precision.HIGHEST)
  h1p_ref[:N, :] = h1 * dinv
  h1p_ref[N:, :] = jnp.zeros((N_PAD - N, H), jnp.float32)
  dinv_ref[...] = dinv


def _tc2(sp_ref, h1p_ref, dinv_ref, b1_ref, g1_ref, be1_ref, w2_ref,
         h2p_ref):
  dinv = dinv_ref[...]
  s = sp_ref[0, :N, :] + sp_ref[1, :N, :] + h1p_ref[:N, :]
  x = jax.nn.relu(dinv * s + b1_ref[...])
  m = jnp.mean(x, axis=0, keepdims=True)
  v = jnp.mean((x - m) ** 2, axis=0, keepdims=True)
  y = (x - m) * lax.rsqrt(v + 1e-5) * g1_ref[...] + be1_ref[...]
  y = jax.nn.relu(y)
  h2 = jnp.dot(y, w2_ref[...], preferred_element_type=jnp.float32,
               precision=lax.Precision.HIGHEST)
  h2p_ref[:N, :] = h2 * dinv
  h2p_ref[N:, :] = jnp.zeros((N_PAD - N, H), jnp.float32)


def _tc3(sp_ref, h2p_ref, dinv_ref, b2_ref, g2_ref, be2_ref, w3_ref, b3_ref,
         out_ref):
  dinv = dinv_ref[...]
  s = sp_ref[0, :N, :] + sp_ref[1, :N, :] + h2p_ref[:N, :]
  x = jax.nn.relu(dinv * s + b2_ref[...])
  m = jnp.mean(x, axis=0, keepdims=True)
  v = jnp.mean((x - m) ** 2, axis=0, keepdims=True)
  y = (x - m) * lax.rsqrt(v + 1e-5) * g2_ref[...] + be2_ref[...]
  z = jnp.dot(y, w3_ref[...], preferred_element_type=jnp.float32,
              precision=lax.Precision.HIGHEST)
  out_ref[...] = jax.nn.relu(z + b3_ref[...])


@jax.jit
def kernel(data, edge_index, W1, b1, g1, be1, W2, b2, g2, be2, W3, b3):
  E = edge_index.shape[1]
  epw = -(-E // (NW * CHUNK)) * CHUNK
  if (epw // CHUNK) % 2:
    epw += CHUNK
  chunks = epw // CHUNK
  e_pad = NW * epw

  row = edge_index[0]
  col = edge_index[1]
  rows3 = jnp.concatenate(
      [row, jnp.zeros((e_pad - E,), row.dtype)]).reshape(NW, chunks, CHUNK)
  cols3 = jnp.concatenate(
      [col, jnp.full((e_pad - E,), N, col.dtype)]).reshape(NW, chunks, CHUNK)

  degp = _make_deg_kernel(chunks)(cols3)

  h1p, dinv = pl.pallas_call(
      _tc1,
      out_shape=[
          jax.ShapeDtypeStruct((N_PAD, H), jnp.float32),
          jax.ShapeDtypeStruct((N, 1), jnp.float32),
      ],
  )(data, W1, degp)

  prop = _make_prop_kernel(chunks)
  s1p = prop(h1p, rows3, cols3)

  h2p = pl.pallas_call(
      _tc2,
      out_shape=jax.ShapeDtypeStruct((N_PAD, H), jnp.float32),
  )(s1p, h1p, dinv, b1.reshape(1, H), g1.reshape(1, H), be1.reshape(1, H), W2)

  s2p = prop(h2p, rows3, cols3)

  out = pl.pallas_call(
      _tc3,
      out_shape=jax.ShapeDtypeStruct((N, C), jnp.float32),
  )(s2p, h2p, dinv, b2.reshape(1, H), g2.reshape(1, H), be2.reshape(1, H),
    W3, b3.reshape(1, C))

  return out


# speedup vs baseline: 34.5449x; 1.0586x over previous
import functools

import jax
import jax.numpy as jnp
from jax import lax
from jax.experimental import pallas as pl
from jax.experimental.pallas import tpu as pltpu
from jax.experimental.pallas import tpu_sc as plsc

N = 10000
D_IN = 128
H = 64
C = 40

NC = 2
NS = 16
NW = NC * NS
CHUNK = 128
N_PAD = 10240
RPT = N_PAD // NS

_mesh = functools.partial(
    plsc.VectorSubcoreMesh, core_axis_name="c", subcore_axis_name="s"
)


def _zero_buf(buf, nrow, width):
  z = jnp.zeros((16,), jnp.float32)

  def body(i, _):
    for k in range(width // 16):
      buf[i, pl.ds(16 * k, 16)] = z
    return 0

  lax.fori_loop(0, nrow, body, 0)


def _make_deg_kernel(chunks):
  @functools.partial(
      pl.kernel,
      out_type=jax.ShapeDtypeStruct((NC, N_PAD, 8), jnp.float32),
      mesh=_mesh(),
      scratch_types=[
          pltpu.VMEM((chunks, CHUNK), jnp.int32),
          pltpu.VMEM((CHUNK, 8), jnp.float32),
          pltpu.VMEM_SHARED((N_PAD, 8), jnp.float32),
          pltpu.SemaphoreType.DMA,
      ],
  )
  def deg_kernel(cols_hbm, const_hbm, out_hbm, cols_v, ones_v, deg_sh, sem):
    cid = lax.axis_index("c")
    sid = lax.axis_index("s")
    wid = sid * NC + cid

    pltpu.sync_copy(const_hbm.at[pl.ds(RPT, CHUNK)], ones_v)
    pltpu.sync_copy(const_hbm.at[pl.ds(0, RPT)],
                    deg_sh.at[pl.ds(sid * RPT, RPT)])
    plsc.subcore_barrier()

    pltpu.sync_copy(cols_hbm.at[wid], cols_v)

    def body(j, _):
      pltpu.sync_copy(ones_v, deg_sh.at[cols_v.at[j]], add=True)
      return 0

    lax.fori_loop(0, chunks, body, 0)
    plsc.subcore_barrier()

    pltpu.sync_copy(deg_sh.at[pl.ds(sid * RPT, RPT)],
                    out_hbm.at[cid, pl.ds(sid * RPT, RPT)])

  return deg_kernel


def _make_prop_kernel(chunks):
  @functools.partial(
      pl.kernel,
      out_type=jax.ShapeDtypeStruct((NC, N_PAD, H), jnp.float32),
      mesh=_mesh(),
      scratch_types=[
          pltpu.VMEM((chunks, CHUNK), jnp.int32),
          pltpu.VMEM((chunks, CHUNK), jnp.int32),
          pltpu.VMEM((CHUNK, H), jnp.float32),
          pltpu.VMEM((CHUNK, H), jnp.float32),
          pltpu.VMEM_SHARED((N_PAD, H), jnp.float32),
          pltpu.VMEM_SHARED((N_PAD, H), jnp.float32),
          pltpu.SemaphoreType.DMA,
          pltpu.SemaphoreType.DMA,
      ],
      compiler_params=pltpu.CompilerParams(use_tc_tiling_on_sc=False),
  )
  def prop_kernel(hp_hbm, rows_hbm, cols_hbm, out_hbm, rows_v, cols_v,
                  buf_a, buf_b, acc_sh, tab_sh, sem_a, sem_b):
    cid = lax.axis_index("c")
    sid = lax.axis_index("s")
    wid = sid * NC + cid

    cp_s = pltpu.async_copy(hp_hbm.at[pl.ds(sid * RPT, RPT)],
                            tab_sh.at[pl.ds(sid * RPT, RPT)], sem_b)
    _zero_buf(buf_a, CHUNK, H)
    for j in range(RPT // CHUNK):
      pltpu.sync_copy(buf_a, acc_sh.at[pl.ds(sid * RPT + j * CHUNK, CHUNK)])
    cp_s.wait()

    @pl.when(sid == NS - 1)
    def _():
      pltpu.sync_copy(buf_a, tab_sh.at[pl.ds(N, CHUNK)])
      pltpu.sync_copy(buf_a.at[pl.ds(0, N_PAD - N - CHUNK)],
                      tab_sh.at[pl.ds(N + CHUNK, N_PAD - N - CHUNK)])
    plsc.subcore_barrier()

    pltpu.sync_copy(rows_hbm.at[wid], rows_v)
    pltpu.sync_copy(cols_hbm.at[wid], cols_v)

    cp_a = pltpu.async_copy(tab_sh.at[rows_v.at[0]], buf_a, sem_a)

    def body(j, _):
      cp_b = pltpu.async_copy(tab_sh.at[rows_v.at[2 * j + 1]], buf_b, sem_b)
      cp_a.wait()
      pltpu.sync_copy(buf_a, acc_sh.at[cols_v.at[2 * j]], add=True)
      pltpu.async_copy(tab_sh.at[rows_v.at[2 * j + 2]], buf_a, sem_a)
      cp_b.wait()
      pltpu.sync_copy(buf_b, acc_sh.at[cols_v.at[2 * j + 1]], add=True)
      return 0

    lax.fori_loop(0, chunks // 2 - 1, body, 0)
    j = chunks - 2
    cp_b = pltpu.async_copy(tab_sh.at[rows_v.at[j + 1]], buf_b, sem_b)
    cp_a.wait()
    pltpu.sync_copy(buf_a, acc_sh.at[cols_v.at[j]], add=True)
    cp_b.wait()
    pltpu.sync_copy(buf_b, acc_sh.at[cols_v.at[j + 1]], add=True)
    plsc.subcore_barrier()

    pltpu.sync_copy(acc_sh.at[pl.ds(sid * RPT, RPT)],
                    out_hbm.at[cid, pl.ds(sid * RPT, RPT)])

  return prop_kernel


def _tc1a(data_ref, w1_ref, h1_ref):
  h1_ref[...] = jnp.dot(data_ref[...], w1_ref[...],
                        preferred_element_type=jnp.float32,
                        precision=lax.Precision.HIGHEST)


GB = 10
GR = N // GB


def _tc1b(h1_ref, degp_ref, h1p_ref, dinv_ref):
  deg = degp_ref[0, :, 0:1] + degp_ref[1, :, 0:1] + 1.0
  dinv = lax.rsqrt(deg)
  h1p_ref[...] = h1_ref[...] * dinv
  dinv_ref[...] = dinv


def _tc2(sp_ref, h1p_ref, dinv_ref, b1_ref, g1_ref, be1_ref, w2_ref,
         h2p_ref):
  dinv = dinv_ref[...]
  s = sp_ref[0, :N, :] + sp_ref[1, :N, :] + h1p_ref[:N, :]
  x = jax.nn.relu(dinv * s + b1_ref[...])
  m = jnp.mean(x, axis=0, keepdims=True)
  v = jnp.mean((x - m) ** 2, axis=0, keepdims=True)
  y = (x - m) * lax.rsqrt(v + 1e-5) * g1_ref[...] + be1_ref[...]
  y = jax.nn.relu(y)
  h2 = jnp.dot(y, w2_ref[...], preferred_element_type=jnp.float32,
               precision=lax.Precision.HIGHEST)
  h2p_ref[:N, :] = h2 * dinv


def _tc3(sp_ref, h2p_ref, dinv_ref, b2_ref, g2_ref, be2_ref, w3_ref, b3_ref,
         out_ref):
  dinv = dinv_ref[...]
  s = sp_ref[0, :N, :] + sp_ref[1, :N, :] + h2p_ref[:N, :]
  x = jax.nn.relu(dinv * s + b2_ref[...])
  m = jnp.mean(x, axis=0, keepdims=True)
  v = jnp.mean((x - m) ** 2, axis=0, keepdims=True)
  y = (x - m) * lax.rsqrt(v + 1e-5) * g2_ref[...] + be2_ref[...]
  z = jnp.dot(y, w3_ref[...], preferred_element_type=jnp.float32,
              precision=lax.Precision.HIGHEST)
  out_ref[...] = jax.nn.relu(z + b3_ref[...])


@jax.jit
def kernel(data, edge_index, W1, b1, g1, be1, W2, b2, g2, be2, W3, b3):
  E = edge_index.shape[1]
  epw = -(-E // (NW * CHUNK)) * CHUNK
  if (epw // CHUNK) % 2:
    epw += CHUNK
  chunks = epw // CHUNK
  e_pad = NW * epw

  ei_p = jnp.pad(edge_index, ((0, 0), (0, e_pad - E)), constant_values=N)
  ei3 = ei_p.reshape(2, NW, chunks, CHUNK)
  rows3 = ei3[0]
  cols3 = ei3[1]

  const8 = jnp.concatenate([
      jnp.zeros((RPT, 8), jnp.float32),
      jnp.ones((CHUNK, 8), jnp.float32),
  ])
  degp = _make_deg_kernel(chunks)(cols3, const8)

  h1 = pl.pallas_call(
      _tc1a,
      out_shape=jax.ShapeDtypeStruct((N, H), jnp.float32),
  )(data, W1)

  h1p, dinv = pl.pallas_call(
      _tc1b,
      grid=(GB,),
      in_specs=[
          pl.BlockSpec((GR, H), lambda i: (i, 0)),
          pl.BlockSpec((2, GR, 8), lambda i: (0, i, 0)),
      ],
      out_specs=[
          pl.BlockSpec((GR, H), lambda i: (i, 0)),
          pl.BlockSpec((GR, 1), lambda i: (i, 0)),
      ],
      out_shape=[
          jax.ShapeDtypeStruct((N_PAD, H), jnp.float32),
          jax.ShapeDtypeStruct((N, 1), jnp.float32),
      ],
  )(h1, degp)

  prop = _make_prop_kernel(chunks)
  s1p = prop(h1p, rows3, cols3)

  h2p = pl.pallas_call(
      _tc2,
      out_shape=jax.ShapeDtypeStruct((N_PAD, H), jnp.float32),
  )(s1p, h1p, dinv, b1.reshape(1, H), g1.reshape(1, H), be1.reshape(1, H), W2)

  s2p = prop(h2p, rows3, cols3)

  out = pl.pallas_call(
      _tc3,
      out_shape=jax.ShapeDtypeStruct((N, C), jnp.float32),
  )(s2p, h2p, dinv, b2.reshape(1, H), g2.reshape(1, H), be2.reshape(1, H),
    W3, b3.reshape(1, C))

  return out


# speedup vs baseline: 34.7788x; 1.0068x over previous
import functools

import jax
import jax.numpy as jnp
from jax import lax
from jax.experimental import pallas as pl
from jax.experimental.pallas import tpu as pltpu
from jax.experimental.pallas import tpu_sc as plsc

N = 10000
D_IN = 128
H = 64
C = 40

NC = 2
NS = 16
NW = NC * NS
CHUNK = 128
N_PAD = 10240
RPT = N_PAD // NS

_mesh = functools.partial(
    plsc.VectorSubcoreMesh, core_axis_name="c", subcore_axis_name="s"
)


def _zero_buf(buf, nrow, width):
  z = jnp.zeros((16,), jnp.float32)

  def body(i, _):
    for k in range(width // 16):
      buf[i, pl.ds(16 * k, 16)] = z
    return 0

  lax.fori_loop(0, nrow, body, 0)


def _make_deg_kernel(chunks):
  @functools.partial(
      pl.kernel,
      out_type=jax.ShapeDtypeStruct((NC, N_PAD, 8), jnp.float32),
      mesh=_mesh(),
      scratch_types=[
          pltpu.VMEM((chunks, CHUNK), jnp.int32),
          pltpu.VMEM((CHUNK, 8), jnp.float32),
          pltpu.VMEM_SHARED((N_PAD, 8), jnp.float32),
          pltpu.SemaphoreType.DMA,
      ],
  )
  def deg_kernel(cols_hbm, const_hbm, out_hbm, cols_v, ones_v, deg_sh, sem):
    cid = lax.axis_index("c")
    sid = lax.axis_index("s")
    wid = sid * NC + cid

    pltpu.sync_copy(const_hbm.at[pl.ds(RPT, CHUNK)], ones_v)
    pltpu.sync_copy(const_hbm.at[pl.ds(0, RPT)],
                    deg_sh.at[pl.ds(sid * RPT, RPT)])
    plsc.subcore_barrier()

    pltpu.sync_copy(cols_hbm.at[wid], cols_v)

    def body(j, _):
      pltpu.sync_copy(ones_v, deg_sh.at[cols_v.at[j]], add=True)
      return 0

    lax.fori_loop(0, chunks, body, 0)
    plsc.subcore_barrier()

    pltpu.sync_copy(deg_sh.at[pl.ds(sid * RPT, RPT)],
                    out_hbm.at[cid, pl.ds(sid * RPT, RPT)])

  return deg_kernel


def _make_prop_kernel(chunks):
  @functools.partial(
      pl.kernel,
      out_type=jax.ShapeDtypeStruct((NC, N_PAD, H), jnp.float32),
      mesh=_mesh(),
      scratch_types=[
          pltpu.VMEM((chunks, CHUNK), jnp.int32),
          pltpu.VMEM((chunks, CHUNK), jnp.int32),
          pltpu.VMEM((CHUNK, H), jnp.float32),
          pltpu.VMEM((CHUNK, H), jnp.float32),
          pltpu.VMEM_SHARED((N_PAD, H), jnp.float32),
          pltpu.VMEM_SHARED((N_PAD, H), jnp.float32),
          pltpu.SemaphoreType.DMA,
          pltpu.SemaphoreType.DMA,
      ],
      compiler_params=pltpu.CompilerParams(use_tc_tiling_on_sc=False),
  )
  def prop_kernel(hp_hbm, rows_hbm, cols_hbm, out_hbm, rows_v, cols_v,
                  buf_a, buf_b, acc_sh, tab_sh, sem_a, sem_b):
    cid = lax.axis_index("c")
    sid = lax.axis_index("s")
    wid = sid * NC + cid

    cp_s = pltpu.async_copy(hp_hbm.at[pl.ds(sid * RPT, RPT)],
                            tab_sh.at[pl.ds(sid * RPT, RPT)], sem_b)
    _zero_buf(buf_a, CHUNK, H)
    for j in range(RPT // CHUNK):
      pltpu.sync_copy(buf_a, acc_sh.at[pl.ds(sid * RPT + j * CHUNK, CHUNK)])
    cp_s.wait()
    plsc.subcore_barrier()

    pltpu.sync_copy(rows_hbm.at[wid], rows_v)
    pltpu.sync_copy(cols_hbm.at[wid], cols_v)

    cp_a = pltpu.async_copy(tab_sh.at[rows_v.at[0]], buf_a, sem_a)

    def body(j, _):
      cp_b = pltpu.async_copy(tab_sh.at[rows_v.at[2 * j + 1]], buf_b, sem_b)
      cp_a.wait()
      pltpu.sync_copy(buf_a, acc_sh.at[cols_v.at[2 * j]], add=True)
      pltpu.async_copy(tab_sh.at[rows_v.at[2 * j + 2]], buf_a, sem_a)
      cp_b.wait()
      pltpu.sync_copy(buf_b, acc_sh.at[cols_v.at[2 * j + 1]], add=True)
      return 0

    lax.fori_loop(0, chunks // 2 - 1, body, 0)
    j = chunks - 2
    cp_b = pltpu.async_copy(tab_sh.at[rows_v.at[j + 1]], buf_b, sem_b)
    cp_a.wait()
    pltpu.sync_copy(buf_a, acc_sh.at[cols_v.at[j]], add=True)
    cp_b.wait()
    pltpu.sync_copy(buf_b, acc_sh.at[cols_v.at[j + 1]], add=True)
    plsc.subcore_barrier()

    pltpu.sync_copy(acc_sh.at[pl.ds(sid * RPT, RPT)],
                    out_hbm.at[cid, pl.ds(sid * RPT, RPT)])

  return prop_kernel


def _tc1a(data_ref, w1_ref, h1_ref):
  h1_ref[...] = jnp.dot(data_ref[...], w1_ref[...],
                        preferred_element_type=jnp.float32,
                        precision=lax.Precision.HIGHEST)


def _tc1b(h1_ref, degp_ref, h1p_ref, dinv_ref):
  deg = degp_ref[0, :N, 0:1] + degp_ref[1, :N, 0:1] + 1.0
  dinv = lax.rsqrt(deg)
  h1p_ref[:N, :] = h1_ref[...] * dinv
  h1p_ref[N:, :] = jnp.zeros((N_PAD - N, H), jnp.float32)
  dinv_ref[...] = dinv


def _tc2(sp_ref, h1p_ref, dinv_ref, b1_ref, g1_ref, be1_ref, w2_ref,
         h2p_ref):
  dinv = dinv_ref[...]
  s = sp_ref[0, :N, :] + sp_ref[1, :N, :] + h1p_ref[:N, :]
  x = jax.nn.relu(dinv * s + b1_ref[...])
  m = jnp.mean(x, axis=0, keepdims=True)
  v = jnp.mean((x - m) ** 2, axis=0, keepdims=True)
  y = (x - m) * lax.rsqrt(v + 1e-5) * g1_ref[...] + be1_ref[...]
  y = jax.nn.relu(y)
  h2 = jnp.dot(y, w2_ref[...], preferred_element_type=jnp.float32,
               precision=lax.Precision.HIGHEST)
  h2p_ref[:N, :] = h2 * dinv
  h2p_ref[N:, :] = jnp.zeros((N_PAD - N, H), jnp.float32)


def _tc3(sp_ref, h2p_ref, dinv_ref, b2_ref, g2_ref, be2_ref, w3_ref, b3_ref,
         out_ref):
  dinv = dinv_ref[...]
  s = sp_ref[0, :N, :] + sp_ref[1, :N, :] + h2p_ref[:N, :]
  x = jax.nn.relu(dinv * s + b2_ref[...])
  m = jnp.mean(x, axis=0, keepdims=True)
  v = jnp.mean((x - m) ** 2, axis=0, keepdims=True)
  y = (x - m) * lax.rsqrt(v + 1e-5) * g2_ref[...] + be2_ref[...]
  z = jnp.dot(y, w3_ref[...], preferred_element_type=jnp.float32,
              precision=lax.Precision.HIGHEST)
  out_ref[...] = jax.nn.relu(z + b3_ref[...])


@jax.jit
def kernel(data, edge_index, W1, b1, g1, be1, W2, b2, g2, be2, W3, b3):
  E = edge_index.shape[1]
  epw = -(-E // (NW * CHUNK)) * CHUNK
  if (epw // CHUNK) % 2:
    epw += CHUNK
  chunks = epw // CHUNK
  e_pad = NW * epw

  ei_p = jnp.pad(edge_index, ((0, 0), (0, e_pad - E)), constant_values=N)
  ei3 = ei_p.reshape(2, NW, chunks, CHUNK)
  rows3 = ei3[0]
  cols3 = ei3[1]

  const8 = jnp.concatenate([
      jnp.zeros((RPT, 8), jnp.float32),
      jnp.ones((CHUNK, 8), jnp.float32),
  ])
  degp = _make_deg_kernel(chunks)(cols3, const8)

  h1 = pl.pallas_call(
      _tc1a,
      out_shape=jax.ShapeDtypeStruct((N, H), jnp.float32),
  )(data, W1)

  h1p, dinv = pl.pallas_call(
      _tc1b,
      out_shape=[
          jax.ShapeDtypeStruct((N_PAD, H), jnp.float32),
          jax.ShapeDtypeStruct((N, 1), jnp.float32),
      ],
  )(h1, degp)

  prop = _make_prop_kernel(chunks)
  s1p = prop(h1p, rows3, cols3)

  h2p = pl.pallas_call(
      _tc2,
      out_shape=jax.ShapeDtypeStruct((N_PAD, H), jnp.float32),
  )(s1p, h1p, dinv, b1.reshape(1, H), g1.reshape(1, H), be1.reshape(1, H), W2)

  s2p = prop(h2p, rows3, cols3)

  out = pl.pallas_call(
      _tc3,
      out_shape=jax.ShapeDtypeStruct((N, C), jnp.float32),
  )(s2p, h2p, dinv, b2.reshape(1, H), g2.reshape(1, H), be2.reshape(1, H),
    W3, b3.reshape(1, C))

  return out
